# initial kernel scaffold (unmeasured)
import jax
import jax.numpy as jnp
from jax import lax
from jax.experimental import pallas as pl
from jax.experimental.pallas import tpu as pltpu

M = 8192
D = 2048
EPS = 1e-6


def kernel(partial, resid, gamma):
    def comm_body(p_ref, recv_ref, send_sem, recv_sem):
        x = lax.axis_index("x")
        y = lax.axis_index("y")
        z = lax.axis_index("z")
        nbr = (x, 1 - y, z)

        bar = pltpu.get_barrier_semaphore()
        pl.semaphore_signal(
            bar, inc=1, device_id=nbr, device_id_type=pl.DeviceIdType.MESH
        )
        pl.semaphore_wait(bar, 1)

        rdma = pltpu.make_async_remote_copy(
            src_ref=p_ref.at[0],
            dst_ref=recv_ref,
            send_sem=send_sem,
            recv_sem=recv_sem,
            device_id=nbr,
            device_id_type=pl.DeviceIdType.MESH,
        )
        rdma.start()
        rdma.wait()

    recv = pl.pallas_call(
        comm_body,
        out_shape=jax.ShapeDtypeStruct((M, D), jnp.float32),
        in_specs=[pl.BlockSpec(memory_space=pltpu.ANY)],
        out_specs=pl.BlockSpec(memory_space=pltpu.ANY),
        scratch_shapes=[pltpu.SemaphoreType.DMA, pltpu.SemaphoreType.DMA],
        compiler_params=pltpu.CompilerParams(collective_id=0),
    )(partial)

    BM = 256
    gamma2 = gamma.reshape(1, D)

    def ln_body(p_ref, r_ref, resid_ref, g_ref, o_ref):
        ysum = p_ref[0] + r_ref[...] + resid_ref[...]
        ms = jnp.mean(ysum * ysum, axis=-1, keepdims=True)
        o_ref[...] = ysum * lax.rsqrt(ms + EPS) * g_ref[...]

    out = pl.pallas_call(
        ln_body,
        grid=(M // BM,),
        in_specs=[
            pl.BlockSpec((1, BM, D), lambda i: (0, i, 0)),
            pl.BlockSpec((BM, D), lambda i: (i, 0)),
            pl.BlockSpec((BM, D), lambda i: (i, 0)),
            pl.BlockSpec((1, D), lambda i: (0, 0)),
        ],
        out_specs=pl.BlockSpec((BM, D), lambda i: (i, 0)),
        out_shape=jax.ShapeDtypeStruct((M, D), jnp.float32),
    )(partial, recv, resid, gamma2)
    return out


# baseline (device time: 436055 ns/iter reference)
import jax
import jax.numpy as jnp
from jax import lax
from jax.experimental import pallas as pl
from jax.experimental.pallas import tpu as pltpu

M = 8192
D = 2048
EPS = 1e-6
HALF = M // 2
S = 16
R = HALF // S
NOUT = 4


def kernel(partial, resid, gamma):
    gamma2 = gamma.reshape(1, D)

    def body(p_ref, resid_ref, g_ref, o_ref,
             recv_buf, p_st, r_st, out_st,
             ysend, yrecv, zsend, zrecv, pin, rin, outcp):
        x = lax.axis_index("x")
        y = lax.axis_index("y")
        z = lax.axis_index("z")
        ynbr = (x, 1 - y, z)
        znbr = (x, y, 1 - z)
        h = jnp.bitwise_xor(y, z)
        mine0 = h * HALF
        theirs0 = (1 - h) * HALF

        bar = pltpu.get_barrier_semaphore()
        pl.semaphore_signal(bar, inc=1, device_id=ynbr,
                            device_id_type=pl.DeviceIdType.MESH)
        pl.semaphore_signal(bar, inc=1, device_id=znbr,
                            device_id_type=pl.DeviceIdType.MESH)
        pl.semaphore_wait(bar, 2)

        y_rdmas = []
        for s in range(S):
            r = pltpu.make_async_remote_copy(
                src_ref=p_ref.at[0, pl.ds(theirs0 + s * R, R), :],
                dst_ref=recv_buf.at[s],
                send_sem=ysend.at[s],
                recv_sem=yrecv.at[s],
                device_id=ynbr,
                device_id_type=pl.DeviceIdType.MESH,
            )
            r.start()
            y_rdmas.append(r)

        def stage_in(s):
            slot = s % 2
            cp_p = pltpu.make_async_copy(
                p_ref.at[0, pl.ds(mine0 + s * R, R), :],
                p_st.at[slot], pin.at[slot])
            cp_r = pltpu.make_async_copy(
                resid_ref.at[pl.ds(mine0 + s * R, R), :],
                r_st.at[slot], rin.at[slot])
            cp_p.start()
            cp_r.start()
            return (cp_p, cp_r)

        pending = {0: stage_in(0)}
        out_cps = {}
        z_rdmas = {}
        for s in range(S):
            slot = s % 2
            oslot = s % NOUT
            if s + 1 < S:
                pending[s + 1] = stage_in(s + 1)
            if s - NOUT >= 0:
                out_cps[s - NOUT].wait()
                z_rdmas[s - NOUT].wait_send()
            cp_p, cp_r = pending.pop(s)
            cp_p.wait()
            cp_r.wait()
            y_rdmas[s].wait_recv()
            ysum = p_st[slot] + recv_buf[s] + r_st[slot]
            ms = jnp.mean(ysum * ysum, axis=-1, keepdims=True)
            out_st[oslot] = ysum * lax.rsqrt(ms + EPS) * g_ref[...]
            cp_o = pltpu.make_async_copy(
                out_st.at[oslot],
                o_ref.at[pl.ds(mine0 + s * R, R), :],
                outcp.at[oslot])
            cp_o.start()
            out_cps[s] = cp_o
            zr = pltpu.make_async_remote_copy(
                src_ref=out_st.at[oslot],
                dst_ref=o_ref.at[pl.ds(mine0 + s * R, R), :],
                send_sem=zsend.at[s],
                recv_sem=zrecv.at[s],
                device_id=znbr,
                device_id_type=pl.DeviceIdType.MESH,
            )
            zr.start()
            z_rdmas[s] = zr

        for s in range(max(0, S - NOUT), S):
            out_cps[s].wait()
            z_rdmas[s].wait_send()
        for s in range(S):
            y_rdmas[s].wait_send()
            zwait = pltpu.make_async_remote_copy(
                src_ref=out_st.at[0],
                dst_ref=o_ref.at[pl.ds(theirs0 + s * R, R), :],
                send_sem=zsend.at[s],
                recv_sem=zrecv.at[s],
                device_id=znbr,
                device_id_type=pl.DeviceIdType.MESH,
            )
            zwait.wait_recv()

    return pl.pallas_call(
        body,
        out_shape=jax.ShapeDtypeStruct((M, D), jnp.float32),
        in_specs=[
            pl.BlockSpec(memory_space=pl.ANY),
            pl.BlockSpec(memory_space=pl.ANY),
            pl.BlockSpec(memory_space=pltpu.VMEM),
        ],
        out_specs=pl.BlockSpec(memory_space=pl.ANY),
        scratch_shapes=[
            pltpu.VMEM((S, R, D), jnp.float32),
            pltpu.VMEM((2, R, D), jnp.float32),
            pltpu.VMEM((2, R, D), jnp.float32),
            pltpu.VMEM((NOUT, R, D), jnp.float32),
            pltpu.SemaphoreType.DMA((S,)),
            pltpu.SemaphoreType.DMA((S,)),
            pltpu.SemaphoreType.DMA((S,)),
            pltpu.SemaphoreType.DMA((S,)),
            pltpu.SemaphoreType.DMA((2,)),
            pltpu.SemaphoreType.DMA((2,)),
            pltpu.SemaphoreType.DMA((NOUT,)),
        ],
        compiler_params=pltpu.CompilerParams(
            collective_id=0,
            vmem_limit_bytes=100 * 1024 * 1024,
        ),
    )(partial, resid, gamma2)
